# Initial kernel scaffold; baseline (speedup 1.0000x reference)
import functools

import jax
import jax.numpy as jnp
from jax import lax
from jax.experimental import pallas as pl
from jax.experimental.pallas import tpu as pltpu
from jax.experimental.pallas import tpu_sc as plsc

NC = 2
NS = 16
NW = NC * NS
CHUNK = 128


@functools.partial(jax.jit, static_argnames=("n_chunks", "d"))
def _gather(table, idx, *, n_chunks, d):
    b_per_w = n_chunks * CHUNK
    total = NW * b_per_w
    mesh = plsc.VectorSubcoreMesh(core_axis_name="c", subcore_axis_name="s")

    @functools.partial(
        pl.kernel,
        mesh=mesh,
        out_type=jax.ShapeDtypeStruct((total, d), jnp.float32),
        scratch_types=[
            pltpu.VMEM((n_chunks, CHUNK), jnp.int32),
            pltpu.VMEM((CHUNK, d), jnp.float32),
            pltpu.SemaphoreType.DMA,
        ],
    )
    def k(table_hbm, idx_hbm, out_hbm, idx_v, rows_v, sem):
        wid = lax.axis_index("s") * NC + lax.axis_index("c")
        base = wid * b_per_w
        pltpu.sync_copy(idx_hbm.at[wid], idx_v)

        def body(j, carry):
            pltpu.async_copy(table_hbm.at[idx_v.at[j]], rows_v, sem).wait()
            pltpu.sync_copy(rows_v, out_hbm.at[pl.ds(base + j * CHUNK, CHUNK)])
            return carry

        lax.fori_loop(0, n_chunks, body, 0)

    return k(table, idx)


def kernel(x, table):
    d = table.shape[1]
    flat = x.reshape(-1).astype(jnp.int32)
    b = flat.shape[0]
    per = NW * CHUNK
    pad = (-b) % per
    if pad:
        flat = jnp.concatenate([flat, jnp.zeros((pad,), jnp.int32)])
    n_chunks = flat.shape[0] // per
    idx = flat.reshape(NW, n_chunks, CHUNK)
    out = _gather(table, idx, n_chunks=n_chunks, d=d)
    if pad:
        out = out[:b]
    return out.reshape(*x.shape, d)


# speedup vs baseline: 5.2228x; 5.2228x over previous
import functools

import jax
import jax.numpy as jnp
from jax import lax
from jax.experimental import pallas as pl
from jax.experimental.pallas import tpu as pltpu
from jax.experimental.pallas import tpu_sc as plsc

NC = 2
NS = 16
NW = NC * NS
CHUNK = 128


@functools.partial(jax.jit, static_argnames=("n_chunks", "d"))
def _gather(table, idx, *, n_chunks, d):
    b_per_w = n_chunks * CHUNK
    total = NW * b_per_w
    mesh = plsc.VectorSubcoreMesh(core_axis_name="c", subcore_axis_name="s")

    @functools.partial(
        pl.kernel,
        mesh=mesh,
        compiler_params=pltpu.CompilerParams(use_tc_tiling_on_sc=False),
        out_type=jax.ShapeDtypeStruct((total, d), jnp.float32),
        scratch_types=[
            pltpu.VMEM((n_chunks, CHUNK), jnp.int32),
            pltpu.VMEM((CHUNK, d), jnp.float32),
            pltpu.SemaphoreType.DMA,
        ],
    )
    def k(table_hbm, idx_hbm, out_hbm, idx_v, rows_v, sem):
        wid = lax.axis_index("s") * NC + lax.axis_index("c")
        base = wid * b_per_w
        pltpu.sync_copy(idx_hbm.at[wid], idx_v)

        def body(j, carry):
            pltpu.async_copy(table_hbm.at[idx_v.at[j]], rows_v, sem).wait()
            pltpu.sync_copy(rows_v, out_hbm.at[pl.ds(base + j * CHUNK, CHUNK)])
            return carry

        lax.fori_loop(0, n_chunks, body, 0)

    return k(table, idx)


def kernel(x, table):
    d = table.shape[1]
    flat = x.reshape(-1).astype(jnp.int32)
    b = flat.shape[0]
    per = NW * CHUNK
    pad = (-b) % per
    if pad:
        flat = jnp.concatenate([flat, jnp.zeros((pad,), jnp.int32)])
    n_chunks = flat.shape[0] // per
    idx = flat.reshape(NW, n_chunks, CHUNK)
    out = _gather(table, idx, n_chunks=n_chunks, d=d)
    if pad:
        out = out[:b]
    return out.reshape(*x.shape, d)


# speedup vs baseline: 6.1998x; 1.1871x over previous
import functools

import jax
import jax.numpy as jnp
from jax import lax
from jax.experimental import pallas as pl
from jax.experimental.pallas import tpu as pltpu
from jax.experimental.pallas import tpu_sc as plsc

NC = 2
NS = 16
NW = NC * NS
CHUNK = 128


NBUF = 4


@functools.partial(jax.jit, static_argnames=("n_chunks", "d"))
def _gather(table, idx, *, n_chunks, d):
    assert n_chunks % NBUF == 0
    n_groups = n_chunks // NBUF
    b_per_w = n_chunks * CHUNK
    total = NW * b_per_w
    mesh = plsc.VectorSubcoreMesh(core_axis_name="c", subcore_axis_name="s")

    @functools.partial(
        pl.kernel,
        mesh=mesh,
        compiler_params=pltpu.CompilerParams(use_tc_tiling_on_sc=False),
        out_type=jax.ShapeDtypeStruct((total, d), jnp.float32),
        scratch_types=[
            pltpu.VMEM((n_chunks, CHUNK), jnp.int32),
            pltpu.VMEM((NBUF, CHUNK, d), jnp.float32),
            [pltpu.SemaphoreType.DMA] * NBUF,
            [pltpu.SemaphoreType.DMA] * NBUF,
        ],
    )
    def k(table_hbm, idx_hbm, out_hbm, idx_v, rows_v, gsems, wsems):
        wid = lax.axis_index("s") * NC + lax.axis_index("c")
        base = wid * b_per_w
        pltpu.sync_copy(idx_hbm.at[wid], idx_v)

        def group(g, carry):
            for b in range(NBUF):
                j = g * NBUF + b

                @pl.when(g > 0)
                def _wait_write(b=b):
                    pltpu.make_async_copy(
                        rows_v.at[b], out_hbm.at[pl.ds(0, CHUNK)], wsems[b]
                    ).wait()

                pltpu.async_copy(table_hbm.at[idx_v.at[j]], rows_v.at[b], gsems[b])
            for b in range(NBUF):
                j = g * NBUF + b
                pltpu.make_async_copy(
                    table_hbm.at[pl.ds(0, CHUNK)], rows_v.at[b], gsems[b]
                ).wait()
                pltpu.async_copy(
                    rows_v.at[b], out_hbm.at[pl.ds(base + j * CHUNK, CHUNK)], wsems[b]
                )
            return carry

        lax.fori_loop(0, n_groups, group, 0)
        for b in range(NBUF):
            pltpu.make_async_copy(
                rows_v.at[b], out_hbm.at[pl.ds(0, CHUNK)], wsems[b]
            ).wait()

    return k(table, idx)


def kernel(x, table):
    d = table.shape[1]
    flat = x.reshape(-1).astype(jnp.int32)
    b = flat.shape[0]
    per = NW * CHUNK
    pad = (-b) % per
    if pad:
        flat = jnp.concatenate([flat, jnp.zeros((pad,), jnp.int32)])
    n_chunks = flat.shape[0] // per
    idx = flat.reshape(NW, n_chunks, CHUNK)
    out = _gather(table, idx, n_chunks=n_chunks, d=d)
    if pad:
        out = out[:b]
    return out.reshape(*x.shape, d)
